# Initial kernel scaffold; baseline (speedup 1.0000x reference)
import functools

import jax
import jax.numpy as jnp
from jax import lax
from jax.experimental import pallas as pl
from jax.experimental.pallas import tpu as pltpu
from jax.experimental.pallas import tpu_sc as plsc

L = 16
CHUNK = 128
NSEG = 10000


def _sc_body(nchunk, nsub, chalf,
             xab, src, tar, xba, out_p, out_a, out_b,
             xbuf, ybuf, gs, gt, sidx, tidx, acc_s, acc_t):
    c = lax.axis_index("c")
    s = lax.axis_index("s")
    c0 = c * chalf
    nk = chalf // L

    rows_sub = NSEG // nsub
    zblk = 125

    def zrow(r, _):
        for k in range(nk):
            xbuf[r, pl.ds(k * L, L)] = jnp.zeros((L,), jnp.float32)
        return 0

    lax.fori_loop(0, zblk, zrow, 0)

    def zcopy(b, _):
        r0 = s * rows_sub + b * zblk
        pltpu.sync_copy(xbuf.at[pl.ds(0, zblk)], acc_s.at[pl.ds(r0, zblk)])
        pltpu.sync_copy(xbuf.at[pl.ds(0, zblk)], acc_t.at[pl.ds(r0, zblk)])
        return 0

    lax.fori_loop(0, rows_sub // zblk, zcopy, 0)
    plsc.subcore_barrier()

    ntrip = (nchunk - s + nsub - 1) // nsub

    def phase_a(t, _):
        base = (s + t * nsub) * CHUNK
        pltpu.sync_copy(xab.at[pl.ds(base, CHUNK), pl.ds(c0, chalf)], xbuf)
        pltpu.sync_copy(xba.at[pl.ds(base, CHUNK), pl.ds(c0, chalf)], ybuf)
        pltpu.sync_copy(src.at[pl.ds(base, CHUNK)], sidx)
        pltpu.sync_copy(tar.at[pl.ds(base, CHUNK)], tidx)

        def erow(r, _):
            for k in range(nk):
                xbuf[r, pl.ds(k * L, L)] = jnp.exp(xbuf[r, pl.ds(k * L, L)])
                ybuf[r, pl.ds(k * L, L)] = jnp.exp(ybuf[r, pl.ds(k * L, L)])
            return 0

        lax.fori_loop(0, CHUNK, erow, 0)
        pltpu.sync_copy(xbuf, acc_s.at[sidx], add=True)
        pltpu.sync_copy(ybuf, acc_t.at[tidx], add=True)
        return 0

    lax.fori_loop(0, ntrip, phase_a, 0)
    plsc.subcore_barrier()

    def rec_block(b, _):
        r0 = s * rows_sub + b * zblk
        pltpu.sync_copy(acc_s.at[pl.ds(r0, zblk)], gs.at[pl.ds(0, zblk)])
        pltpu.sync_copy(acc_t.at[pl.ds(r0, zblk)], gt.at[pl.ds(0, zblk)])

        def rrow(r, _):
            for k in range(nk):
                gs[r, pl.ds(k * L, L)] = 1.0 / gs[r, pl.ds(k * L, L)]
                gt[r, pl.ds(k * L, L)] = 1.0 / gt[r, pl.ds(k * L, L)]
            return 0

        lax.fori_loop(0, zblk, rrow, 0)
        pltpu.sync_copy(gs.at[pl.ds(0, zblk)], acc_s.at[pl.ds(r0, zblk)])
        pltpu.sync_copy(gt.at[pl.ds(0, zblk)], acc_t.at[pl.ds(r0, zblk)])
        return 0

    lax.fori_loop(0, rows_sub // zblk, rec_block, 0)
    plsc.subcore_barrier()

    def phase_b(t, _):
        base = (s + t * nsub) * CHUNK
        pltpu.sync_copy(xab.at[pl.ds(base, CHUNK), pl.ds(c0, chalf)], xbuf)
        pltpu.sync_copy(xba.at[pl.ds(base, CHUNK), pl.ds(c0, chalf)], ybuf)
        pltpu.sync_copy(src.at[pl.ds(base, CHUNK)], sidx)
        pltpu.sync_copy(tar.at[pl.ds(base, CHUNK)], tidx)
        pltpu.sync_copy(acc_s.at[sidx], gs)
        pltpu.sync_copy(acc_t.at[tidx], gt)

        def orow(r, _):
            for k in range(nk):
                za = jnp.exp(xbuf[r, pl.ds(k * L, L)]) * gs[r, pl.ds(k * L, L)]
                zb = jnp.exp(ybuf[r, pl.ds(k * L, L)]) * gt[r, pl.ds(k * L, L)]
                xbuf[r, pl.ds(k * L, L)] = za
                ybuf[r, pl.ds(k * L, L)] = zb
                gs[r, pl.ds(k * L, L)] = za * zb
            return 0

        lax.fori_loop(0, CHUNK, orow, 0)
        pltpu.sync_copy(gs, out_p.at[pl.ds(base, CHUNK), pl.ds(c0, chalf)])
        pltpu.sync_copy(xbuf, out_a.at[pl.ds(base, CHUNK), pl.ds(c0, chalf)])
        pltpu.sync_copy(ybuf, out_b.at[pl.ds(base, CHUNK), pl.ds(c0, chalf)])
        return 0

    lax.fori_loop(0, ntrip, phase_b, 0)


def kernel(xab, src_id, tar_id, xba):
    E, C = xab.shape
    info = plsc.get_sparse_core_info()
    nc, ns = info.num_cores, info.num_subcores
    chalf = C // nc
    nchunk = E // CHUNK
    mesh = plsc.VectorSubcoreMesh(core_axis_name="c", subcore_axis_name="s")
    out_type = (jax.ShapeDtypeStruct((E, C), jnp.float32),) * 3
    f = pl.kernel(
        functools.partial(_sc_body, nchunk, ns, chalf),
        out_type=out_type,
        mesh=mesh,
        scratch_types=[
            pltpu.VMEM((CHUNK, chalf), jnp.float32),
            pltpu.VMEM((CHUNK, chalf), jnp.float32),
            pltpu.VMEM((CHUNK, chalf), jnp.float32),
            pltpu.VMEM((CHUNK, chalf), jnp.float32),
            pltpu.VMEM((CHUNK,), jnp.int32),
            pltpu.VMEM((CHUNK,), jnp.int32),
            pltpu.VMEM_SHARED((NSEG, chalf), jnp.float32),
            pltpu.VMEM_SHARED((NSEG, chalf), jnp.float32),
        ],
    )
    return f(xab, src_id, tar_id, xba)


# speedup vs baseline: 2.9023x; 2.9023x over previous
import functools

import jax
import jax.numpy as jnp
from jax import lax
from jax.experimental import pallas as pl
from jax.experimental.pallas import tpu as pltpu
from jax.experimental.pallas import tpu_sc as plsc

L = 16
CHUNK = 128
NSEG = 10000


def _sc_body(nchunk, nsub, chalf,
             xab, src, tar, xba, out_p, out_a, out_b,
             xbuf, ybuf, gs, gt, sidx, tidx, acc_s, acc_t):
    c = lax.axis_index("c")
    s = lax.axis_index("s")
    c0 = c * chalf
    nk = chalf // L

    rows_sub = NSEG // nsub
    zblk = 125

    def zrow(r, _):
        for k in range(nk):
            xbuf[r, pl.ds(k * L, L)] = jnp.zeros((L,), jnp.float32)
        return 0

    lax.fori_loop(0, zblk, zrow, 0)

    def zcopy(b, _):
        r0 = s * rows_sub + b * zblk
        pltpu.sync_copy(xbuf.at[pl.ds(0, zblk)], acc_s.at[pl.ds(r0, zblk)])
        pltpu.sync_copy(xbuf.at[pl.ds(0, zblk)], acc_t.at[pl.ds(r0, zblk)])
        return 0

    lax.fori_loop(0, rows_sub // zblk, zcopy, 0)
    plsc.subcore_barrier()

    ntrip = (nchunk - s + nsub - 1) // nsub

    def phase_a(t, _):
        base = (s + t * nsub) * CHUNK
        pltpu.sync_copy(xab.at[pl.ds(base, CHUNK), pl.ds(c0, chalf)], xbuf)
        pltpu.sync_copy(xba.at[pl.ds(base, CHUNK), pl.ds(c0, chalf)], ybuf)
        pltpu.sync_copy(src.at[pl.ds(base, CHUNK)], sidx)
        pltpu.sync_copy(tar.at[pl.ds(base, CHUNK)], tidx)

        def erow(r, _):
            for k in range(nk):
                xbuf[r, pl.ds(k * L, L)] = jnp.exp(xbuf[r, pl.ds(k * L, L)])
                ybuf[r, pl.ds(k * L, L)] = jnp.exp(ybuf[r, pl.ds(k * L, L)])
            return 0

        lax.fori_loop(0, CHUNK, erow, 0)
        pltpu.sync_copy(xbuf, acc_s.at[sidx], add=True)
        pltpu.sync_copy(ybuf, acc_t.at[tidx], add=True)
        return 0

    lax.fori_loop(0, ntrip, phase_a, 0)
    plsc.subcore_barrier()

    def rec_block(b, _):
        r0 = s * rows_sub + b * zblk
        pltpu.sync_copy(acc_s.at[pl.ds(r0, zblk)], gs.at[pl.ds(0, zblk)])
        pltpu.sync_copy(acc_t.at[pl.ds(r0, zblk)], gt.at[pl.ds(0, zblk)])

        def rrow(r, _):
            for k in range(nk):
                gs[r, pl.ds(k * L, L)] = 1.0 / gs[r, pl.ds(k * L, L)]
                gt[r, pl.ds(k * L, L)] = 1.0 / gt[r, pl.ds(k * L, L)]
            return 0

        lax.fori_loop(0, zblk, rrow, 0)
        pltpu.sync_copy(gs.at[pl.ds(0, zblk)], acc_s.at[pl.ds(r0, zblk)])
        pltpu.sync_copy(gt.at[pl.ds(0, zblk)], acc_t.at[pl.ds(r0, zblk)])
        return 0

    lax.fori_loop(0, rows_sub // zblk, rec_block, 0)
    plsc.subcore_barrier()

    def phase_b(t, _):
        base = (s + t * nsub) * CHUNK
        pltpu.sync_copy(xab.at[pl.ds(base, CHUNK), pl.ds(c0, chalf)], xbuf)
        pltpu.sync_copy(xba.at[pl.ds(base, CHUNK), pl.ds(c0, chalf)], ybuf)
        pltpu.sync_copy(src.at[pl.ds(base, CHUNK)], sidx)
        pltpu.sync_copy(tar.at[pl.ds(base, CHUNK)], tidx)
        pltpu.sync_copy(acc_s.at[sidx], gs)
        pltpu.sync_copy(acc_t.at[tidx], gt)

        def orow(r, _):
            for k in range(nk):
                za = jnp.exp(xbuf[r, pl.ds(k * L, L)]) * gs[r, pl.ds(k * L, L)]
                zb = jnp.exp(ybuf[r, pl.ds(k * L, L)]) * gt[r, pl.ds(k * L, L)]
                xbuf[r, pl.ds(k * L, L)] = za
                ybuf[r, pl.ds(k * L, L)] = zb
                gs[r, pl.ds(k * L, L)] = za * zb
            return 0

        lax.fori_loop(0, CHUNK, orow, 0)
        pltpu.sync_copy(gs, out_p.at[pl.ds(base, CHUNK), pl.ds(c0, chalf)])
        pltpu.sync_copy(xbuf, out_a.at[pl.ds(base, CHUNK), pl.ds(c0, chalf)])
        pltpu.sync_copy(ybuf, out_b.at[pl.ds(base, CHUNK), pl.ds(c0, chalf)])
        return 0

    lax.fori_loop(0, ntrip, phase_b, 0)


def kernel(xab, src_id, tar_id, xba):
    E, C = xab.shape
    info = plsc.get_sparse_core_info()
    nc, ns = info.num_cores, info.num_subcores
    chalf = C // nc
    nchunk = E // CHUNK
    mesh = plsc.VectorSubcoreMesh(core_axis_name="c", subcore_axis_name="s")
    out_type = (jax.ShapeDtypeStruct((E, C), jnp.float32),) * 3
    f = pl.kernel(
        functools.partial(_sc_body, nchunk, ns, chalf),
        out_type=out_type,
        mesh=mesh,
        compiler_params=pltpu.CompilerParams(use_tc_tiling_on_sc=False),
        scratch_types=[
            pltpu.VMEM((CHUNK, chalf), jnp.float32),
            pltpu.VMEM((CHUNK, chalf), jnp.float32),
            pltpu.VMEM((CHUNK, chalf), jnp.float32),
            pltpu.VMEM((CHUNK, chalf), jnp.float32),
            pltpu.VMEM((CHUNK,), jnp.int32),
            pltpu.VMEM((CHUNK,), jnp.int32),
            pltpu.VMEM_SHARED((NSEG, chalf), jnp.float32),
            pltpu.VMEM_SHARED((NSEG, chalf), jnp.float32),
        ],
    )
    return f(xab, src_id, tar_id, xba)


# speedup vs baseline: 7.1346x; 2.4582x over previous
import functools

import jax
import jax.numpy as jnp
from jax import lax
from jax.experimental import pallas as pl
from jax.experimental.pallas import tpu as pltpu
from jax.experimental.pallas import tpu_sc as plsc

L = 16
CHUNK = 128
NSEG = 10000


def _sc_body(nchunk, nsub, cwidth, chalf,
             xab, src, tar, xba, out_p, out_a, out_b,
             xb0, xb1, yb0, yb1, oa0, oa1, ob0, ob1, op0, op1,
             ga0, ga1, gb0, gb1,
             si0, si1, si2, si3, ti0, ti1, ti2, ti3,
             in_s0, in_s1, out_s0, out_s1, g_s0, g_s1,
             ix_s0, ix_s1, ix_s2, ix_s3,
             acc_s, acc_t):
    c = lax.axis_index("c")
    s = lax.axis_index("s")
    nk = chalf // L
    xb = (xb0, xb1)
    yb = (yb0, yb1)
    oa = (oa0, oa1)
    ob = (ob0, ob1)
    op = (op0, op1)
    ga = (ga0, ga1)
    gb = (gb0, gb1)
    si = (si0, si1, si2, si3)
    ti = (ti0, ti1, ti2, ti3)
    in_s = (in_s0, in_s1)
    out_s = (out_s0, out_s1)
    g_s = (g_s0, g_s1)
    ix_s = (ix_s0, ix_s1, ix_s2, ix_s3)

    for sub in range(cwidth // chalf):
        _sweep(nchunk, nsub, chalf, c * cwidth + sub * chalf, s, nk,
               xab, src, tar, xba, out_p, out_a, out_b,
               xb, yb, oa, ob, op, ga, gb, si, ti,
               in_s, out_s, g_s, ix_s, acc_s, acc_t)


def _sweep(nchunk, nsub, chalf, c0, s, nk,
           xab, src, tar, xba, out_p, out_a, out_b,
           xb, yb, oa, ob, op, ga, gb, si, ti,
           in_s, out_s, g_s, ix_s, acc_s, acc_t):
    xb0, ga0, gb0 = xb[0], ga[0], gb[0]

    def xslice(ref, base):
        return ref.at[pl.ds(base, CHUNK), pl.ds(c0, chalf)]

    def chunk_base(t):
        return (s + t * nsub) * CHUNK

    def fire_in(t, b, q):
        base = chunk_base(t)
        pltpu.async_copy(xslice(xab, base), xb[b], in_s[b])
        pltpu.async_copy(xslice(xba, base), yb[b], in_s[b])
        pltpu.async_copy(src.at[pl.ds(base, CHUNK)], si[q], ix_s[q])
        pltpu.async_copy(tar.at[pl.ds(base, CHUNK)], ti[q], ix_s[q])

    def wait_in(b):
        pltpu.make_async_copy(xslice(xab, 0), xb[b], in_s[b]).wait()
        pltpu.make_async_copy(xslice(xba, 0), yb[b], in_s[b]).wait()

    def wait_idx(q):
        pltpu.make_async_copy(src.at[pl.ds(0, CHUNK)], si[q], ix_s[q]).wait()
        pltpu.make_async_copy(tar.at[pl.ds(0, CHUNK)], ti[q], ix_s[q]).wait()

    rows_sub = NSEG // nsub
    zblk = 125

    @plsc.parallel_loop(0, zblk)
    def _(r):
        for k in range(nk):
            xb0[r, pl.ds(k * L, L)] = jnp.zeros((L,), jnp.float32)

    def zcopy(b, _):
        r0 = s * rows_sub + b * zblk
        pltpu.sync_copy(xb0.at[pl.ds(0, zblk)], acc_s.at[pl.ds(r0, zblk)])
        pltpu.sync_copy(xb0.at[pl.ds(0, zblk)], acc_t.at[pl.ds(r0, zblk)])
        return 0

    lax.fori_loop(0, rows_sub // zblk, zcopy, 0)
    plsc.subcore_barrier()

    ntrip = (nchunk - s + nsub - 1) // nsub

    fire_in(0, 0, 0)
    fire_in(1, 1, 1)

    def phase_a(tt, _):
        for q in range(4):
            b = q % 2
            t = tt * 4 + q

            @pl.when(t < ntrip)
            def _():
                wait_in(b)
                wait_idx(q)

                @pl.when(t >= 2)
                def _():
                    pltpu.make_async_copy(oa[b], acc_s.at[si[q]], out_s[b]).wait()
                    pltpu.make_async_copy(ob[b], acc_t.at[ti[q]], out_s[b]).wait()

                @plsc.parallel_loop(0, CHUNK, unroll=2)
                def _(r):
                    for k in range(nk):
                        oa[b][r, pl.ds(k * L, L)] = jnp.exp(xb[b][r, pl.ds(k * L, L)])
                        ob[b][r, pl.ds(k * L, L)] = jnp.exp(yb[b][r, pl.ds(k * L, L)])

                pltpu.async_copy(oa[b], acc_s.at[si[q]], out_s[b], add=True)
                pltpu.async_copy(ob[b], acc_t.at[ti[q]], out_s[b], add=True)

                @pl.when(t + 2 < ntrip)
                def _():
                    fire_in(t + 2, b, (q + 2) % 4)
        return 0

    lax.fori_loop(0, (ntrip + 3) // 4, phase_a, 0)
    for b in range(2):
        pltpu.make_async_copy(oa[b], acc_s.at[si[0]], out_s[b]).wait()
        pltpu.make_async_copy(ob[b], acc_t.at[ti[0]], out_s[b]).wait()
    plsc.subcore_barrier()

    def rec_block(blk, _):
        r0 = s * rows_sub + blk * zblk
        pltpu.sync_copy(acc_s.at[pl.ds(r0, zblk)], ga0.at[pl.ds(0, zblk)])
        pltpu.sync_copy(acc_t.at[pl.ds(r0, zblk)], gb0.at[pl.ds(0, zblk)])

        @plsc.parallel_loop(0, zblk)
        def _(r):
            for k in range(nk):
                ga0[r, pl.ds(k * L, L)] = 1.0 / ga0[r, pl.ds(k * L, L)]
                gb0[r, pl.ds(k * L, L)] = 1.0 / gb0[r, pl.ds(k * L, L)]

        pltpu.sync_copy(ga0.at[pl.ds(0, zblk)], acc_s.at[pl.ds(r0, zblk)])
        pltpu.sync_copy(gb0.at[pl.ds(0, zblk)], acc_t.at[pl.ds(r0, zblk)])
        return 0

    lax.fori_loop(0, rows_sub // zblk, rec_block, 0)
    plsc.subcore_barrier()

    fire_in(0, 0, 0)
    fire_in(1, 1, 1)
    wait_idx(0)
    pltpu.async_copy(acc_s.at[si[0]], ga[0], g_s[0])
    pltpu.async_copy(acc_t.at[ti[0]], gb[0], g_s[0])

    def phase_b(tt, _):
        for q in range(4):
            b = q % 2
            t = tt * 4 + q

            @pl.when(t < ntrip)
            def _():
                @pl.when(t + 1 < ntrip)
                def _():
                    wait_idx((q + 1) % 4)
                    pltpu.async_copy(acc_s.at[si[(q + 1) % 4]], ga[1 - b], g_s[1 - b])
                    pltpu.async_copy(acc_t.at[ti[(q + 1) % 4]], gb[1 - b], g_s[1 - b])

                wait_in(b)
                pltpu.make_async_copy(acc_s.at[si[q]], ga[b], g_s[b]).wait()
                pltpu.make_async_copy(acc_t.at[ti[q]], gb[b], g_s[b]).wait()

                @pl.when(t >= 2)
                def _():
                    base0 = chunk_base(t)
                    pltpu.make_async_copy(op[b], xslice(out_p, base0), out_s[b]).wait()
                    pltpu.make_async_copy(oa[b], xslice(out_a, base0), out_s[b]).wait()
                    pltpu.make_async_copy(ob[b], xslice(out_b, base0), out_s[b]).wait()

                @plsc.parallel_loop(0, CHUNK, unroll=2)
                def _(r):
                    for k in range(nk):
                        za = jnp.exp(xb[b][r, pl.ds(k * L, L)]) * ga[b][r, pl.ds(k * L, L)]
                        zb_ = jnp.exp(yb[b][r, pl.ds(k * L, L)]) * gb[b][r, pl.ds(k * L, L)]
                        oa[b][r, pl.ds(k * L, L)] = za
                        ob[b][r, pl.ds(k * L, L)] = zb_
                        op[b][r, pl.ds(k * L, L)] = za * zb_

                base = chunk_base(t)
                pltpu.async_copy(op[b], xslice(out_p, base), out_s[b])
                pltpu.async_copy(oa[b], xslice(out_a, base), out_s[b])
                pltpu.async_copy(ob[b], xslice(out_b, base), out_s[b])

                @pl.when(t + 2 < ntrip)
                def _():
                    fire_in(t + 2, b, (q + 2) % 4)
        return 0

    lax.fori_loop(0, (ntrip + 3) // 4, phase_b, 0)
    for b in range(2):
        pltpu.make_async_copy(op[b], xslice(out_p, 0), out_s[b]).wait()
        pltpu.make_async_copy(oa[b], xslice(out_a, 0), out_s[b]).wait()
        pltpu.make_async_copy(ob[b], xslice(out_b, 0), out_s[b]).wait()
    plsc.subcore_barrier()


def kernel(xab, src_id, tar_id, xba):
    E, C = xab.shape
    info = plsc.get_sparse_core_info()
    nc, ns = info.num_cores, info.num_subcores
    cwidth = C // nc
    chalf = cwidth // 2
    nchunk = E // CHUNK
    mesh = plsc.VectorSubcoreMesh(core_axis_name="c", subcore_axis_name="s")
    out_type = (jax.ShapeDtypeStruct((E, C), jnp.float32),) * 3
    buf = pltpu.VMEM((CHUNK, chalf), jnp.float32)
    ibuf = pltpu.VMEM((CHUNK,), jnp.int32)
    f = pl.kernel(
        functools.partial(_sc_body, nchunk, ns, cwidth, chalf),
        out_type=out_type,
        mesh=mesh,
        compiler_params=pltpu.CompilerParams(use_tc_tiling_on_sc=False),
        scratch_types=(
            [buf] * 14
            + [ibuf] * 8
            + [pltpu.SemaphoreType.DMA] * 10
            + [
                pltpu.VMEM_SHARED((NSEG, chalf), jnp.float32),
                pltpu.VMEM_SHARED((NSEG, chalf), jnp.float32),
            ]
        ),
    )
    return f(xab, src_id, tar_id, xba)


# speedup vs baseline: 7.2038x; 1.0097x over previous
import functools

import jax
import jax.numpy as jnp
from jax import lax
from jax.experimental import pallas as pl
from jax.experimental.pallas import tpu as pltpu
from jax.experimental.pallas import tpu_sc as plsc

L = 16
CHUNK = 128
NSEG = 10000


def _sc_body(nchunk, nsub, cwidth, chalf,
             xab, src, tar, xba, out_p, out_a, out_b,
             xb0, xb1, yb0, yb1, oa0, oa1, ob0, ob1, op0, op1,
             ga0, ga1, gb0, gb1,
             si0, si1, si2, si3, ti0, ti1, ti2, ti3,
             in_s0, in_s1, out_s0, out_s1, g_s0, g_s1,
             ix_s0, ix_s1, ix_s2, ix_s3,
             acc_s, acc_t):
    c = lax.axis_index("c")
    s = lax.axis_index("s")
    nk = chalf // L
    xb = (xb0, xb1)
    yb = (yb0, yb1)
    oa = (oa0, oa1)
    ob = (ob0, ob1)
    op = (op0, op1)
    ga = (ga0, ga1)
    gb = (gb0, gb1)
    si = (si0, si1, si2, si3)
    ti = (ti0, ti1, ti2, ti3)
    in_s = (in_s0, in_s1)
    out_s = (out_s0, out_s1)
    g_s = (g_s0, g_s1)
    ix_s = (ix_s0, ix_s1, ix_s2, ix_s3)

    for sub in range(cwidth // chalf):
        _sweep(nchunk, nsub, chalf, c * cwidth + sub * chalf, s, nk,
               xab, src, tar, xba, out_p, out_a, out_b,
               xb, yb, oa, ob, op, ga, gb, si, ti,
               in_s, out_s, g_s, ix_s, acc_s, acc_t)


def _sweep(nchunk, nsub, chalf, c0, s, nk,
           xab, src, tar, xba, out_p, out_a, out_b,
           xb, yb, oa, ob, op, ga, gb, si, ti,
           in_s, out_s, g_s, ix_s, acc_s, acc_t):
    xb0, ga0, gb0 = xb[0], ga[0], gb[0]

    def xslice(ref, base):
        return ref.at[pl.ds(base, CHUNK), pl.ds(c0, chalf)]

    def chunk_base(t):
        return (s + t * nsub) * CHUNK

    def fire_in(t, b, q):
        base = chunk_base(t)
        pltpu.async_copy(xslice(xab, base), xb[b], in_s[b])
        pltpu.async_copy(xslice(xba, base), yb[b], in_s[b])
        pltpu.async_copy(src.at[pl.ds(base, CHUNK)], si[q], ix_s[q])
        pltpu.async_copy(tar.at[pl.ds(base, CHUNK)], ti[q], ix_s[q])

    def wait_in(b):
        pltpu.make_async_copy(xslice(xab, 0), xb[b], in_s[b]).wait()
        pltpu.make_async_copy(xslice(xba, 0), yb[b], in_s[b]).wait()

    def wait_idx(q):
        pltpu.make_async_copy(src.at[pl.ds(0, CHUNK)], si[q], ix_s[q]).wait()
        pltpu.make_async_copy(tar.at[pl.ds(0, CHUNK)], ti[q], ix_s[q]).wait()

    rows_sub = NSEG // nsub
    zblk = 125

    @plsc.parallel_loop(0, zblk)
    def _(r):
        for k in range(nk):
            xb0[r, pl.ds(k * L, L)] = jnp.zeros((L,), jnp.float32)

    def zcopy(b, _):
        r0 = s * rows_sub + b * zblk
        pltpu.sync_copy(xb0.at[pl.ds(0, zblk)], acc_s.at[pl.ds(r0, zblk)])
        pltpu.sync_copy(xb0.at[pl.ds(0, zblk)], acc_t.at[pl.ds(r0, zblk)])
        return 0

    lax.fori_loop(0, rows_sub // zblk, zcopy, 0)
    plsc.subcore_barrier()

    ntrip = (nchunk - s + nsub - 1) // nsub

    _scope_a = jax.named_scope("phase_a")
    _scope_a.__enter__()
    fire_in(0, 0, 0)
    fire_in(1, 1, 1)

    def phase_a(tt, _):
        for q in range(4):
            b = q % 2
            t = tt * 4 + q

            @pl.when(t < ntrip)
            def _():
                wait_in(b)
                wait_idx(q)

                @pl.when(t >= 2)
                def _():
                    pltpu.make_async_copy(oa[b], acc_s.at[si[q]], out_s[b]).wait()
                    pltpu.make_async_copy(ob[b], acc_t.at[ti[q]], out_s[b]).wait()

                @plsc.parallel_loop(0, CHUNK, unroll=4)
                def _(r):
                    for k in range(nk):
                        oa[b][r, pl.ds(k * L, L)] = jnp.exp(xb[b][r, pl.ds(k * L, L)])
                        ob[b][r, pl.ds(k * L, L)] = jnp.exp(yb[b][r, pl.ds(k * L, L)])

                pltpu.async_copy(oa[b], acc_s.at[si[q]], out_s[b], add=True)
                pltpu.async_copy(ob[b], acc_t.at[ti[q]], out_s[b], add=True)

                @pl.when(t + 2 < ntrip)
                def _():
                    fire_in(t + 2, b, (q + 2) % 4)
        return 0

    lax.fori_loop(0, (ntrip + 3) // 4, phase_a, 0)
    for b in range(2):
        pltpu.make_async_copy(oa[b], acc_s.at[si[0]], out_s[b]).wait()
        pltpu.make_async_copy(ob[b], acc_t.at[ti[0]], out_s[b]).wait()
    plsc.subcore_barrier()
    _scope_a.__exit__(None, None, None)

    def rec_block(blk, _):
        r0 = s * rows_sub + blk * zblk
        pltpu.sync_copy(acc_s.at[pl.ds(r0, zblk)], ga0.at[pl.ds(0, zblk)])
        pltpu.sync_copy(acc_t.at[pl.ds(r0, zblk)], gb0.at[pl.ds(0, zblk)])

        @plsc.parallel_loop(0, zblk)
        def _(r):
            for k in range(nk):
                ga0[r, pl.ds(k * L, L)] = 1.0 / ga0[r, pl.ds(k * L, L)]
                gb0[r, pl.ds(k * L, L)] = 1.0 / gb0[r, pl.ds(k * L, L)]

        pltpu.sync_copy(ga0.at[pl.ds(0, zblk)], acc_s.at[pl.ds(r0, zblk)])
        pltpu.sync_copy(gb0.at[pl.ds(0, zblk)], acc_t.at[pl.ds(r0, zblk)])
        return 0

    lax.fori_loop(0, rows_sub // zblk, rec_block, 0)
    plsc.subcore_barrier()

    _scope_b = jax.named_scope("phase_b")
    _scope_b.__enter__()
    fire_in(0, 0, 0)
    fire_in(1, 1, 1)
    wait_idx(0)
    pltpu.async_copy(acc_s.at[si[0]], ga[0], g_s[0])
    pltpu.async_copy(acc_t.at[ti[0]], gb[0], g_s[0])

    def phase_b(tt, _):
        for q in range(4):
            b = q % 2
            t = tt * 4 + q

            @pl.when(t < ntrip)
            def _():
                @pl.when(t + 1 < ntrip)
                def _():
                    wait_idx((q + 1) % 4)
                    pltpu.async_copy(acc_s.at[si[(q + 1) % 4]], ga[1 - b], g_s[1 - b])
                    pltpu.async_copy(acc_t.at[ti[(q + 1) % 4]], gb[1 - b], g_s[1 - b])

                wait_in(b)
                pltpu.make_async_copy(acc_s.at[si[q]], ga[b], g_s[b]).wait()
                pltpu.make_async_copy(acc_t.at[ti[q]], gb[b], g_s[b]).wait()

                @pl.when(t >= 2)
                def _():
                    base0 = chunk_base(t)
                    pltpu.make_async_copy(op[b], xslice(out_p, base0), out_s[b]).wait()
                    pltpu.make_async_copy(oa[b], xslice(out_a, base0), out_s[b]).wait()
                    pltpu.make_async_copy(ob[b], xslice(out_b, base0), out_s[b]).wait()

                @plsc.parallel_loop(0, CHUNK, unroll=4)
                def _(r):
                    for k in range(nk):
                        za = jnp.exp(xb[b][r, pl.ds(k * L, L)]) * ga[b][r, pl.ds(k * L, L)]
                        zb_ = jnp.exp(yb[b][r, pl.ds(k * L, L)]) * gb[b][r, pl.ds(k * L, L)]
                        oa[b][r, pl.ds(k * L, L)] = za
                        ob[b][r, pl.ds(k * L, L)] = zb_
                        op[b][r, pl.ds(k * L, L)] = za * zb_

                base = chunk_base(t)
                pltpu.async_copy(op[b], xslice(out_p, base), out_s[b])
                pltpu.async_copy(oa[b], xslice(out_a, base), out_s[b])
                pltpu.async_copy(ob[b], xslice(out_b, base), out_s[b])

                @pl.when(t + 2 < ntrip)
                def _():
                    fire_in(t + 2, b, (q + 2) % 4)
        return 0

    lax.fori_loop(0, (ntrip + 3) // 4, phase_b, 0)
    for b in range(2):
        pltpu.make_async_copy(op[b], xslice(out_p, 0), out_s[b]).wait()
        pltpu.make_async_copy(oa[b], xslice(out_a, 0), out_s[b]).wait()
        pltpu.make_async_copy(ob[b], xslice(out_b, 0), out_s[b]).wait()
    plsc.subcore_barrier()
    _scope_b.__exit__(None, None, None)


def kernel(xab, src_id, tar_id, xba):
    E, C = xab.shape
    info = plsc.get_sparse_core_info()
    nc, ns = info.num_cores, info.num_subcores
    cwidth = C // nc
    chalf = cwidth // 2
    nchunk = E // CHUNK
    mesh = plsc.VectorSubcoreMesh(core_axis_name="c", subcore_axis_name="s")
    out_type = (jax.ShapeDtypeStruct((E, C), jnp.float32),) * 3
    buf = pltpu.VMEM((CHUNK, chalf), jnp.float32)
    ibuf = pltpu.VMEM((CHUNK,), jnp.int32)
    f = pl.kernel(
        functools.partial(_sc_body, nchunk, ns, cwidth, chalf),
        out_type=out_type,
        mesh=mesh,
        compiler_params=pltpu.CompilerParams(use_tc_tiling_on_sc=False),
        scratch_types=(
            [buf] * 14
            + [ibuf] * 8
            + [pltpu.SemaphoreType.DMA] * 10
            + [
                pltpu.VMEM_SHARED((NSEG, chalf), jnp.float32),
                pltpu.VMEM_SHARED((NSEG, chalf), jnp.float32),
            ]
        ),
    )
    return f(xab, src_id, tar_id, xba)
